# baseline (speedup 1.0000x reference)
import functools

import jax
import jax.numpy as jnp
from jax import lax
from jax.experimental import pallas as pl
from jax.experimental.pallas import tpu as pltpu
from jax.experimental.pallas import tpu_sc as plsc

N, D, H, E = 32768, 768, 768, 8
BLKA = 1024
SEG = 512
P = 2 * N + E * SEG
NB_C = P // SEG
L = 16
NC, NS = 2, 16
NW = NC * NS
B_PER_W = N // NW
G = 16
N_G = B_PER_W // G


def _route_block(x_ref, wg_ref, bg_ref,
                 e0_ref, e1_ref, w0_ref, w1_ref, r0_ref, r1_ref, cnt_ref,
                 carry):
    i = pl.program_id(0)

    @pl.when(i == 0)
    def _():
        carry[...] = jnp.zeros((1, E), jnp.int32)

    xb16 = x_ref[...].astype(jnp.bfloat16)
    logits = jnp.dot(xb16, wg_ref[...],
                     preferred_element_type=jnp.float32) + bg_ref[...]
    m = jnp.max(logits, axis=-1, keepdims=True)
    ex = jnp.exp(logits - m)
    gw = ex / jnp.sum(ex, axis=-1, keepdims=True)

    iota = lax.broadcasted_iota(jnp.int32, gw.shape, 1)
    w1 = jnp.max(gw, axis=-1, keepdims=True)
    i1 = jnp.min(jnp.where(gw == w1, iota, E), axis=-1, keepdims=True)
    sel1 = iota == i1
    w_rest = jnp.where(sel1, -jnp.inf, gw)
    w2 = jnp.max(w_rest, axis=-1, keepdims=True)
    i2 = jnp.min(jnp.where(w_rest == w2, iota, E), axis=-1, keepdims=True)
    sel2 = iota == i2

    def _cumsum0(a):
        s = a
        k = 1
        while k < a.shape[0]:
            z = jnp.zeros((k, a.shape[1]), a.dtype)
            s = s + jnp.concatenate([z, s[:a.shape[0] - k]], axis=0)
            k *= 2
        return s

    c = carry[...]
    o1 = sel1.astype(jnp.int32)
    o2 = sel2.astype(jnp.int32)
    cum1 = _cumsum0(o1)
    r0 = jnp.sum(o1 * (cum1 - 1 + c), axis=1)
    tot1 = cum1[-1:, :]
    cum2 = _cumsum0(o2)
    r1 = jnp.sum(o2 * (cum2 - 1 + c + tot1), axis=1)
    newc = c + tot1 + cum2[-1:, :]
    carry[...] = newc

    shp = (BLKA // 128, 128)
    e0_ref[...] = i1.reshape(shp)
    e1_ref[...] = i2.reshape(shp)
    w0_ref[...] = w1.reshape(shp)
    w1_ref[...] = w2.reshape(shp)
    r0_ref[...] = r0.reshape(shp)
    r1_ref[...] = r1.reshape(shp)
    cnt_ref[...] = jnp.broadcast_to(newc, (8, E))


def _route(x, wg16, bg2):
    nb = N // BLKA
    rshape = (N // 128, 128)
    blk = pl.BlockSpec((BLKA // 128, 128), lambda i: (i, 0))
    out_shapes = [
        jax.ShapeDtypeStruct(rshape, jnp.int32),
        jax.ShapeDtypeStruct(rshape, jnp.int32),
        jax.ShapeDtypeStruct(rshape, jnp.float32),
        jax.ShapeDtypeStruct(rshape, jnp.float32),
        jax.ShapeDtypeStruct(rshape, jnp.int32),
        jax.ShapeDtypeStruct(rshape, jnp.int32),
        jax.ShapeDtypeStruct((8, E), jnp.int32),
    ]
    return pl.pallas_call(
        _route_block,
        grid=(nb,),
        in_specs=[
            pl.BlockSpec((BLKA, D), lambda i: (i, 0)),
            pl.BlockSpec((D, E), lambda i: (0, 0)),
            pl.BlockSpec((1, E), lambda i: (0, 0)),
        ],
        out_specs=[blk, blk, blk, blk, blk, blk,
                   pl.BlockSpec((8, E), lambda i: (0, 0))],
        out_shape=out_shapes,
        scratch_shapes=[pltpu.VMEM((1, E), jnp.int32)],
    )(x, wg16, bg2)


def _make_dispatch():
    mesh = plsc.VectorSubcoreMesh(core_axis_name="c", subcore_axis_name="s", num_cores=NC, num_subcores=NS)

    @functools.partial(
        pl.kernel,
        out_type=jax.ShapeDtypeStruct((P, D), jnp.float32),
        mesh=mesh,
        scratch_types=[
            pltpu.VMEM((N_G, G), jnp.int32),
            pltpu.VMEM((N_G, G), jnp.int32),
            pltpu.VMEM((2, G, D), jnp.float32),
            pltpu.SemaphoreType.DMA,
            pltpu.SemaphoreType.DMA,
            pltpu.SemaphoreType.DMA,
        ],
    )
    def dispatch(x_hbm, pos0_hbm, pos1_hbm, xs_hbm,
                 pos0_v, pos1_v, row_v, sem_in, sem0, sem1):
        wid = lax.axis_index("s") * NC + lax.axis_index("c")
        base = wid * B_PER_W
        pltpu.sync_copy(pos0_hbm.at[pl.ds(wid * N_G, N_G)], pos0_v)
        pltpu.sync_copy(pos1_hbm.at[pl.ds(wid * N_G, N_G)], pos1_v)

        cp = pltpu.async_copy(x_hbm.at[pl.ds(base, G)], row_v.at[0], sem_in)
        cp.wait()

        def body(g, _):
            buf = lax.rem(g, 2)
            nxt = lax.rem(g + 1, 2)

            @pl.when(g + 1 < N_G)
            def _():
                pltpu.async_copy(x_hbm.at[pl.ds(base + (g + 1) * G, G)],
                                 row_v.at[nxt], sem_in)
            c0 = pltpu.async_copy(row_v.at[buf], xs_hbm.at[pos0_v.at[g]],
                                  sem0)
            c1 = pltpu.async_copy(row_v.at[buf], xs_hbm.at[pos1_v.at[g]],
                                  sem1)
            c0.wait()
            c1.wait()

            @pl.when(g + 1 < N_G)
            def _():
                pltpu.make_async_copy(x_hbm.at[pl.ds(base, G)],
                                      row_v.at[nxt], sem_in).wait()
            return ()
        lax.fori_loop(0, N_G, body, ())

    return dispatch


def _seg_matmul(bexp_ref, xs_ref, we_ref, be_ref, ys_ref):
    ys_ref[...] = jnp.dot(xs_ref[...].astype(jnp.bfloat16), we_ref[0],
                          preferred_element_type=jnp.float32) + be_ref[0]


def _expert_mm(xs, we16, be, block_e):
    grid_spec = pltpu.PrefetchScalarGridSpec(
        num_scalar_prefetch=1,
        grid=(NB_C,),
        in_specs=[
            pl.BlockSpec((SEG, D), lambda i, bexp: (i, 0)),
            pl.BlockSpec((1, D, H), lambda i, bexp: (bexp[i], 0, 0)),
            pl.BlockSpec((1, 1, H), lambda i, bexp: (bexp[i], 0, 0)),
        ],
        out_specs=pl.BlockSpec((SEG, H), lambda i, bexp: (i, 0)),
    )
    return pl.pallas_call(
        _seg_matmul,
        grid_spec=grid_spec,
        out_shape=jax.ShapeDtypeStruct((P, H), jnp.float32),
    )(block_e, xs, we16, be.reshape(E, 1, H))


def _make_combine():
    mesh = plsc.VectorSubcoreMesh(core_axis_name="c", subcore_axis_name="s", num_cores=NC, num_subcores=NS)
    nchunk = H // L

    @functools.partial(
        pl.kernel,
        out_type=jax.ShapeDtypeStruct((N, H), jnp.float32),
        mesh=mesh,
        scratch_types=[
            pltpu.VMEM((G, L), jnp.float32),
            pltpu.VMEM((G, L), jnp.float32),
            pltpu.VMEM((N_G, G), jnp.int32),
            pltpu.VMEM((N_G, G), jnp.int32),
            pltpu.VMEM((2, G, H), jnp.float32),
            pltpu.VMEM((2, G, H), jnp.float32),
            pltpu.VMEM((G, H), jnp.float32),
            pltpu.SemaphoreType.DMA,
            pltpu.SemaphoreType.DMA,
        ],
    )
    def combine(ys_hbm, pos0_hbm, pos1_hbm, w0b_hbm, w1b_hbm, y_hbm,
                w0_v, w1_v, pos0_v, pos1_v,
                a_v, b_v, o_v, sem0, sem1):
        wid = lax.axis_index("s") * NC + lax.axis_index("c")
        base = wid * B_PER_W
        pltpu.sync_copy(pos0_hbm.at[pl.ds(wid * N_G, N_G)], pos0_v)
        pltpu.sync_copy(pos1_hbm.at[pl.ds(wid * N_G, N_G)], pos1_v)

        pltpu.async_copy(ys_hbm.at[pos0_v.at[0]], a_v.at[0], sem0)
        pltpu.async_copy(ys_hbm.at[pos1_v.at[0]], b_v.at[0], sem1)
        pltpu.make_async_copy(ys_hbm.at[pos0_v.at[0]], a_v.at[0], sem0).wait()
        pltpu.make_async_copy(ys_hbm.at[pos1_v.at[0]], b_v.at[0], sem1).wait()

        def body(g, _):
            buf = lax.rem(g, 2)
            nxt = lax.rem(g + 1, 2)

            @pl.when(g + 1 < N_G)
            def _():
                pltpu.async_copy(ys_hbm.at[pos0_v.at[g + 1]], a_v.at[nxt],
                                 sem0)
                pltpu.async_copy(ys_hbm.at[pos1_v.at[g + 1]], b_v.at[nxt],
                                 sem1)

            pltpu.sync_copy(w0b_hbm.at[pl.ds(base + g * G, G)], w0_v)
            pltpu.sync_copy(w1b_hbm.at[pl.ds(base + g * G, G)], w1_v)

            def row_body(j, _):
                wj0 = w0_v[j, :]
                wj1 = w1_v[j, :]

                def col_body(cc, _):
                    sl = pl.ds(cc * L, L)
                    o_v[j, sl] = (wj0 * a_v[buf, j, sl]
                                  + wj1 * b_v[buf, j, sl])
                    return ()
                lax.fori_loop(0, nchunk, col_body, ())
                return ()
            lax.fori_loop(0, G, row_body, ())
            pltpu.sync_copy(o_v, y_hbm.at[pl.ds(base + g * G, G)])

            @pl.when(g + 1 < N_G)
            def _():
                pltpu.make_async_copy(ys_hbm.at[pos0_v.at[0]], a_v.at[nxt],
                                      sem0).wait()
                pltpu.make_async_copy(ys_hbm.at[pos1_v.at[0]], b_v.at[nxt],
                                      sem1).wait()
            return ()
        lax.fori_loop(0, N_G, body, ())

    return combine


_make_dispatch = functools.cache(_make_dispatch)
_make_combine = functools.cache(_make_combine)


@jax.jit
def kernel(x, Wg, bg, We, be):
    wg16 = Wg.astype(jnp.bfloat16)
    we16 = We.astype(jnp.bfloat16)
    bg2 = bg.reshape(1, E)

    e0, e1, w0, w1, r0, r1, cnt = _route(x, wg16, bg2)
    counts = cnt[0]
    padded = ((counts + SEG - 1) // SEG) * SEG
    csum = jnp.cumsum(padded)
    offs = jnp.concatenate([jnp.zeros((1,), jnp.int32),
                            csum[:-1]]).astype(jnp.int32)
    block_e = jnp.repeat(jnp.arange(E, dtype=jnp.int32), padded // SEG,
                         total_repeat_length=NB_C)

    e0f, e1f = e0.reshape(N), e1.reshape(N)
    r0f, r1f = r0.reshape(N), r1.reshape(N)
    pos0 = (offs[e0f] + r0f).reshape(NW * N_G, G)
    pos1 = (offs[e1f] + r1f).reshape(NW * N_G, G)
    w0b = jnp.broadcast_to(w0.reshape(N)[:, None], (N, L))
    w1b = jnp.broadcast_to(w1.reshape(N)[:, None], (N, L))

    xs = _make_dispatch()(x, pos0, pos1)
    ys = _expert_mm(xs, we16, be, block_e)
    y = _make_combine()(ys, pos0, pos1, w0b, w1b)
    return y


# speedup vs baseline: 1.3444x; 1.3444x over previous
import functools

import jax
import jax.numpy as jnp
from jax import lax
from jax.experimental import pallas as pl
from jax.experimental.pallas import tpu as pltpu
from jax.experimental.pallas import tpu_sc as plsc

N, D, H, E = 32768, 768, 768, 8
BLKA = 1024
SEG = 512
P = 2 * N + E * SEG
NB_C = P // SEG
NC, NS = 2, 16
NW = NC * NS
B_PER_W = N // NW
G = 16
N_G = B_PER_W // G


def _route_block(x_ref, wg_ref, bg_ref,
                 e0_ref, e1_ref, w0_ref, w1_ref, r0_ref, r1_ref, cnt_ref,
                 carry):
    i = pl.program_id(0)

    @pl.when(i == 0)
    def _():
        carry[...] = jnp.zeros((1, E), jnp.int32)

    xb16 = x_ref[...].astype(jnp.bfloat16)
    logits = jnp.dot(xb16, wg_ref[...],
                     preferred_element_type=jnp.float32) + bg_ref[...]
    m = jnp.max(logits, axis=-1, keepdims=True)
    ex = jnp.exp(logits - m)
    gw = ex / jnp.sum(ex, axis=-1, keepdims=True)

    iota = lax.broadcasted_iota(jnp.int32, gw.shape, 1)
    w1 = jnp.max(gw, axis=-1, keepdims=True)
    i1 = jnp.min(jnp.where(gw == w1, iota, E), axis=-1, keepdims=True)
    sel1 = iota == i1
    w_rest = jnp.where(sel1, -jnp.inf, gw)
    w2 = jnp.max(w_rest, axis=-1, keepdims=True)
    i2 = jnp.min(jnp.where(w_rest == w2, iota, E), axis=-1, keepdims=True)
    sel2 = iota == i2

    def _cumsum0(a):
        s = a
        k = 1
        while k < a.shape[0]:
            z = jnp.zeros((k, a.shape[1]), a.dtype)
            s = s + jnp.concatenate([z, s[:a.shape[0] - k]], axis=0)
            k *= 2
        return s

    c = carry[...]
    o1 = sel1.astype(jnp.int32)
    o2 = sel2.astype(jnp.int32)
    cum1 = _cumsum0(o1)
    r0 = jnp.sum(o1 * (cum1 - 1 + c), axis=1)
    tot1 = cum1[-1:, :]
    cum2 = _cumsum0(o2)
    r1 = jnp.sum(o2 * (cum2 - 1 + c + tot1), axis=1)
    newc = c + tot1 + cum2[-1:, :]
    carry[...] = newc

    shp = (BLKA // 128, 128)
    e0_ref[...] = i1.reshape(shp)
    e1_ref[...] = i2.reshape(shp)
    w0_ref[...] = w1.reshape(shp)
    w1_ref[...] = w2.reshape(shp)
    r0_ref[...] = r0.reshape(shp)
    r1_ref[...] = r1.reshape(shp)
    cnt_ref[...] = jnp.broadcast_to(newc, (8, E))


def _route(x, wg16, bg2):
    nb = N // BLKA
    rshape = (N // 128, 128)
    blk = pl.BlockSpec((BLKA // 128, 128), lambda i: (i, 0))
    out_shapes = [
        jax.ShapeDtypeStruct(rshape, jnp.int32),
        jax.ShapeDtypeStruct(rshape, jnp.int32),
        jax.ShapeDtypeStruct(rshape, jnp.float32),
        jax.ShapeDtypeStruct(rshape, jnp.float32),
        jax.ShapeDtypeStruct(rshape, jnp.int32),
        jax.ShapeDtypeStruct(rshape, jnp.int32),
        jax.ShapeDtypeStruct((8, E), jnp.int32),
    ]
    return pl.pallas_call(
        _route_block,
        grid=(nb,),
        in_specs=[
            pl.BlockSpec((BLKA, D), lambda i: (i, 0)),
            pl.BlockSpec((D, E), lambda i: (0, 0)),
            pl.BlockSpec((1, E), lambda i: (0, 0)),
        ],
        out_specs=[blk, blk, blk, blk, blk, blk,
                   pl.BlockSpec((8, E), lambda i: (0, 0))],
        out_shape=out_shapes,
        scratch_shapes=[pltpu.VMEM((1, E), jnp.int32)],
    )(x, wg16, bg2)


def _make_dispatch():
    mesh = plsc.VectorSubcoreMesh(core_axis_name="c", subcore_axis_name="s",
                                  num_cores=NC, num_subcores=NS)

    @functools.partial(
        pl.kernel,
        out_type=jax.ShapeDtypeStruct((P, D), jnp.float32),
        mesh=mesh,
        scratch_types=[
            pltpu.VMEM((N_G, G), jnp.int32),
            pltpu.VMEM((N_G, G), jnp.int32),
            pltpu.VMEM((2, G, D), jnp.float32),
            pltpu.SemaphoreType.DMA,
            pltpu.SemaphoreType.DMA,
            pltpu.SemaphoreType.DMA,
        ],
    )
    def dispatch(x_hbm, pos0_hbm, pos1_hbm, xs_hbm,
                 pos0_v, pos1_v, row_v, sem_in, sem0, sem1):
        wid = lax.axis_index("s") * NC + lax.axis_index("c")
        base = wid * B_PER_W
        pltpu.sync_copy(pos0_hbm.at[pl.ds(wid * N_G, N_G)], pos0_v)
        pltpu.sync_copy(pos1_hbm.at[pl.ds(wid * N_G, N_G)], pos1_v)

        cp = pltpu.async_copy(x_hbm.at[pl.ds(base, G)], row_v.at[0], sem_in)
        cp.wait()

        def body(g, _):
            buf = lax.rem(g, 2)
            nxt = lax.rem(g + 1, 2)

            @pl.when(g + 1 < N_G)
            def _():
                pltpu.async_copy(x_hbm.at[pl.ds(base + (g + 1) * G, G)],
                                 row_v.at[nxt], sem_in)
            c0 = pltpu.async_copy(row_v.at[buf], xs_hbm.at[pos0_v.at[g]],
                                  sem0)
            c1 = pltpu.async_copy(row_v.at[buf], xs_hbm.at[pos1_v.at[g]],
                                  sem1)
            c0.wait()
            c1.wait()

            @pl.when(g + 1 < N_G)
            def _():
                pltpu.make_async_copy(x_hbm.at[pl.ds(base, G)],
                                      row_v.at[nxt], sem_in).wait()
            return ()
        lax.fori_loop(0, N_G, body, ())

    return dispatch


def _seg_matmul(bexp_ref, xs_ref, we_ref, be_ref, ys_ref):
    ys_ref[...] = jnp.dot(xs_ref[...].astype(jnp.bfloat16), we_ref[0],
                          preferred_element_type=jnp.float32) + be_ref[0]


def _expert_mm(xs, we16, be, block_e):
    grid_spec = pltpu.PrefetchScalarGridSpec(
        num_scalar_prefetch=1,
        grid=(NB_C,),
        in_specs=[
            pl.BlockSpec((SEG, D), lambda i, bexp: (i, 0)),
            pl.BlockSpec((1, D, H), lambda i, bexp: (bexp[i], 0, 0)),
            pl.BlockSpec((1, 1, H), lambda i, bexp: (bexp[i], 0, 0)),
        ],
        out_specs=pl.BlockSpec((SEG, H), lambda i, bexp: (i, 0)),
    )
    return pl.pallas_call(
        _seg_matmul,
        grid_spec=grid_spec,
        out_shape=jax.ShapeDtypeStruct((P, H), jnp.float32),
    )(block_e, xs, we16, be.reshape(E, 1, H))


def _make_gather():
    mesh = plsc.VectorSubcoreMesh(core_axis_name="c", subcore_axis_name="s",
                                  num_cores=NC, num_subcores=NS)

    @functools.partial(
        pl.kernel,
        out_type=[jax.ShapeDtypeStruct((N, H), jnp.float32),
                  jax.ShapeDtypeStruct((N, H), jnp.float32)],
        mesh=mesh,
        scratch_types=[
            pltpu.VMEM((N_G, G), jnp.int32),
            pltpu.VMEM((N_G, G), jnp.int32),
            pltpu.VMEM((2, G, H), jnp.float32),
            pltpu.VMEM((2, G, H), jnp.float32),
            pltpu.SemaphoreType.DMA,
            pltpu.SemaphoreType.DMA,
        ],
    )
    def gather(ys_hbm, pos0_hbm, pos1_hbm, a_hbm, b_hbm,
               pos0_v, pos1_v, a_v, b_v, sem0, sem1):
        wid = lax.axis_index("s") * NC + lax.axis_index("c")
        base = wid * B_PER_W
        pltpu.sync_copy(pos0_hbm.at[pl.ds(wid * N_G, N_G)], pos0_v)
        pltpu.sync_copy(pos1_hbm.at[pl.ds(wid * N_G, N_G)], pos1_v)

        pltpu.async_copy(ys_hbm.at[pos0_v.at[0]], a_v.at[0], sem0)
        pltpu.async_copy(ys_hbm.at[pos1_v.at[0]], b_v.at[0], sem1)
        pltpu.make_async_copy(ys_hbm.at[pos0_v.at[0]], a_v.at[0], sem0).wait()
        pltpu.make_async_copy(ys_hbm.at[pos1_v.at[0]], b_v.at[0], sem1).wait()

        def body(g, _):
            buf = lax.rem(g, 2)
            nxt = lax.rem(g + 1, 2)

            @pl.when(g + 1 < N_G)
            def _():
                pltpu.async_copy(ys_hbm.at[pos0_v.at[g + 1]], a_v.at[nxt],
                                 sem0)
                pltpu.async_copy(ys_hbm.at[pos1_v.at[g + 1]], b_v.at[nxt],
                                 sem1)

            sl = pl.ds(base + g * G, G)
            pltpu.sync_copy(a_v.at[buf], a_hbm.at[sl])
            pltpu.sync_copy(b_v.at[buf], b_hbm.at[sl])

            @pl.when(g + 1 < N_G)
            def _():
                pltpu.make_async_copy(ys_hbm.at[pos0_v.at[0]], a_v.at[nxt],
                                      sem0).wait()
                pltpu.make_async_copy(ys_hbm.at[pos1_v.at[0]], b_v.at[nxt],
                                      sem1).wait()
            return ()
        lax.fori_loop(0, N_G, body, ())

    return gather


def _combine_block(a_ref, b_ref, w0_ref, w1_ref, y_ref):
    y_ref[...] = w0_ref[...] * a_ref[...] + w1_ref[...] * b_ref[...]


def _combine(a, b, w0c, w1c):
    nb = N // SEG
    blk = pl.BlockSpec((SEG, H), lambda i: (i, 0))
    wblk = pl.BlockSpec((SEG, 1), lambda i: (i, 0))
    return pl.pallas_call(
        _combine_block,
        grid=(nb,),
        in_specs=[blk, blk, wblk, wblk],
        out_specs=blk,
        out_shape=jax.ShapeDtypeStruct((N, H), jnp.float32),
    )(a, b, w0c, w1c)


_make_dispatch = functools.cache(_make_dispatch)
_make_gather = functools.cache(_make_gather)


@jax.jit
def kernel(x, Wg, bg, We, be):
    wg16 = Wg.astype(jnp.bfloat16)
    we16 = We.astype(jnp.bfloat16)
    bg2 = bg.reshape(1, E)

    e0, e1, w0, w1, r0, r1, cnt = _route(x, wg16, bg2)
    counts = cnt[0]
    padded = ((counts + SEG - 1) // SEG) * SEG
    csum = jnp.cumsum(padded)
    offs = jnp.concatenate([jnp.zeros((1,), jnp.int32),
                            csum[:-1]]).astype(jnp.int32)
    block_e = jnp.repeat(jnp.arange(E, dtype=jnp.int32), padded // SEG,
                         total_repeat_length=NB_C)

    e0f, e1f = e0.reshape(N), e1.reshape(N)
    r0f, r1f = r0.reshape(N), r1.reshape(N)
    pos0 = (offs[e0f] + r0f).reshape(NW * N_G, G)
    pos1 = (offs[e1f] + r1f).reshape(NW * N_G, G)

    xs = _make_dispatch()(x, pos0, pos1)
    ys = _expert_mm(xs, we16, be, block_e)
    a, b = _make_gather()(ys, pos0, pos1)
    y = _combine(a, b, w0.reshape(N, 1), w1.reshape(N, 1))
    return y
